# baseline (speedup 1.0000x reference)
import functools

import jax
import jax.numpy as jnp
from jax import lax
from jax.experimental import pallas as pl
from jax.experimental.pallas import tpu as pltpu
from jax.experimental.pallas import tpu_sc as plsc

B = 16384
D = 64
FU = 16
FI = 16

_info = plsc.get_sparse_core_info()
_NC, _NS = _info.num_cores, _info.num_subcores
_NW = _NC * _NS
_BPW = B // _NW

_mesh = plsc.VectorSubcoreMesh(core_axis_name="c", subcore_axis_name="s")


@functools.partial(
    pl.kernel,
    mesh=_mesh,
    out_type=(
        jax.ShapeDtypeStruct((B, D), jnp.float32),
        jax.ShapeDtypeStruct((B, D), jnp.float32),
    ),
    scratch_types=[
        pltpu.VMEM((_BPW,), jnp.int32),
        pltpu.VMEM((_BPW,), jnp.int32),
        pltpu.VMEM((_BPW, D), jnp.float32),
        pltpu.VMEM((_BPW, D), jnp.float32),
        pltpu.SemaphoreType.DMA,
        pltpu.SemaphoreType.DMA,
    ],
    compiler_params=pltpu.CompilerParams(use_tc_tiling_on_sc=False),
)
def _sc_gather(ut_hbm, it_hbm, uid_hbm, iid_hbm, uout_hbm, iout_hbm,
               uidx_v, iidx_v, urows_v, irows_v, usem, isem):
    wid = lax.axis_index("s") * _NC + lax.axis_index("c")
    base = wid * _BPW
    pltpu.sync_copy(uid_hbm.at[pl.ds(base, _BPW)], uidx_v)
    pltpu.sync_copy(iid_hbm.at[pl.ds(base, _BPW)], iidx_v)
    ucp = pltpu.async_copy(ut_hbm.at[uidx_v], urows_v, usem)
    icp = pltpu.async_copy(it_hbm.at[iidx_v], irows_v, isem)
    ucp.wait()
    icp.wait()
    pltpu.sync_copy(urows_v, uout_hbm.at[pl.ds(base, _BPW)])
    pltpu.sync_copy(irows_v, iout_hbm.at[pl.ds(base, _BPW)])


def _dense_body(ur, uf, ir, if_, wue, wuf, bu, wie, wif, bi,
                gu, betau, gi, betai, uo, io):
    def tower(rows, feats, we, wf, b, g, beta, out):
        x = (jnp.dot(rows[...], we[...], preferred_element_type=jnp.float32)
             + jnp.dot(feats[...], wf[...], preferred_element_type=jnp.float32)
             + b[...])
        mu = jnp.mean(x, axis=-1, keepdims=True)
        xc = x - mu
        var = jnp.mean(xc * xc, axis=-1, keepdims=True)
        out[...] = xc * lax.rsqrt(var + 1e-5) * g[...] + beta[...]

    tower(ur, uf, wue, wuf, bu, gu, betau, uo)
    tower(ir, if_, wie, wif, bi, gi, betai, io)


_BLK = 2048


def _dense(u_rows, u_feats, i_rows, i_feats, Wue, Wuf, bu, Wie, Wif, bi,
           gu, beta_u, gi, beta_i):
    grid = (B // _BLK,)
    row_spec = pl.BlockSpec((_BLK, D), lambda i: (i, 0))
    ufeat_spec = pl.BlockSpec((_BLK, FU), lambda i: (i, 0))
    ifeat_spec = pl.BlockSpec((_BLK, FI), lambda i: (i, 0))
    full = lambda shape: pl.BlockSpec(shape, lambda i: (0, 0))
    return pl.pallas_call(
        _dense_body,
        grid=grid,
        in_specs=[
            row_spec, ufeat_spec, row_spec, ifeat_spec,
            full((D, D)), full((FU, D)), full((1, D)),
            full((D, D)), full((FI, D)), full((1, D)),
            full((1, D)), full((1, D)), full((1, D)), full((1, D)),
        ],
        out_specs=[row_spec, row_spec],
        out_shape=[
            jax.ShapeDtypeStruct((B, D), jnp.float32),
            jax.ShapeDtypeStruct((B, D), jnp.float32),
        ],
    )(u_rows, u_feats, i_rows, i_feats, Wue, Wuf, bu, Wie, Wif, bi,
      gu, beta_u, gi, beta_i)


def kernel(user_ids, item_ids, user_feats, item_feats, user_table, item_table,
           Wu, bu, Wi, bi, gu, beta_u, gi, beta_i):
    uid = user_ids.astype(jnp.int32)
    iid = item_ids.astype(jnp.int32)
    u_rows, i_rows = _sc_gather(user_table, item_table, uid, iid)
    u_emb, i_emb = _dense(
        u_rows, user_feats, i_rows, item_feats,
        Wu[:D], Wu[D:], bu.reshape(1, D),
        Wi[:D], Wi[D:], bi.reshape(1, D),
        gu.reshape(1, D), beta_u.reshape(1, D),
        gi.reshape(1, D), beta_i.reshape(1, D),
    )
    return (u_emb, i_emb)
